# Initial kernel scaffold; baseline (speedup 1.0000x reference)
import jax
import jax.numpy as jnp
from jax.experimental import pallas as pl


def kernel(x, edge_index, group, W_conv1, b_conv1, W_lin1, b_lin1, W_conv2, b_conv2, W_lin2, b_lin2, W_head, b_head):
    raise NotImplementedError("write your pallas kernel here")


# speedup vs baseline: 10.1777x; 10.1777x over previous
import functools
import jax
import jax.numpy as jnp
from jax import lax
from jax.experimental import pallas as pl
from jax.experimental.pallas import tpu as pltpu
from jax.experimental.pallas import tpu_sc as plsc

NC = 2
NS = 16
NW = NC * NS
LANES = 16
C = 128


def _sc_mesh():
    return plsc.VectorSubcoreMesh(
        core_axis_name="c", subcore_axis_name="s", num_cores=NC,
        num_subcores=NS)


def _make_degree_kernel(n_acc, chunks, d):
    rows_per_tile = n_acc // NS

    @functools.partial(
        pl.kernel,
        mesh=_sc_mesh(),
        out_type=jax.ShapeDtypeStruct((NC, n_acc, d), jnp.float32),
        scratch_types=[
            pltpu.VMEM_SHARED((n_acc, d), jnp.float32),
            pltpu.VMEM((chunks, C), jnp.int32),
            pltpu.VMEM((C, d), jnp.float32),
        ],
    )
    def deg_kernel(dst_hbm, zeros_hbm, ones_hbm, out_hbm, acc_sh, dstv, onesv):
        c = lax.axis_index("c")
        s = lax.axis_index("s")
        w = c * NS + s
        pltpu.sync_copy(ones_hbm, onesv)
        pltpu.sync_copy(dst_hbm.at[w], dstv)
        sl = pl.ds(s * rows_per_tile, rows_per_tile)
        pltpu.sync_copy(zeros_hbm.at[sl], acc_sh.at[sl])
        plsc.subcore_barrier()

        def step(i, carry):
            pltpu.sync_copy(onesv, acc_sh.at[dstv.at[i]], add=True)
            return carry

        lax.fori_loop(0, chunks, step, 0)
        plsc.subcore_barrier()
        pltpu.sync_copy(acc_sh.at[sl], out_hbm.at[c, sl])

    return deg_kernel


def _make_mp_kernel(n_acc, chunks, d):
    rows_per_tile = n_acc // NS

    @functools.partial(
        pl.kernel,
        mesh=_sc_mesh(),
        out_type=jax.ShapeDtypeStruct((NC, n_acc, d), jnp.float32),
        scratch_types=[
            pltpu.VMEM_SHARED((n_acc, d), jnp.float32),
            pltpu.VMEM((chunks, C), jnp.int32),
            pltpu.VMEM((chunks, C), jnp.int32),
            pltpu.VMEM((C, d), jnp.float32),
        ],
    )
    def mp_kernel(y_hbm, src_hbm, dst_hbm, zeros_hbm, out_hbm,
                  acc_sh, srcv, dstv, rowsv):
        c = lax.axis_index("c")
        s = lax.axis_index("s")
        w = c * NS + s
        pltpu.sync_copy(src_hbm.at[w], srcv)
        pltpu.sync_copy(dst_hbm.at[w], dstv)
        sl = pl.ds(s * rows_per_tile, rows_per_tile)
        pltpu.sync_copy(zeros_hbm.at[sl], acc_sh.at[sl])
        plsc.subcore_barrier()

        def step(i, carry):
            pltpu.sync_copy(y_hbm.at[srcv.at[i]], rowsv)
            pltpu.sync_copy(rowsv, acc_sh.at[dstv.at[i]], add=True)
            return carry

        lax.fori_loop(0, chunks, step, 0)
        plsc.subcore_barrier()
        pltpu.sync_copy(acc_sh.at[sl], out_hbm.at[c, sl])

    return mp_kernel


def _tc_call(body, out_shapes):
    return pl.pallas_call(body, out_shape=out_shapes)


def _mm1_body(x_ref, wc_ref, wl_ref, bl_ref, xw_ref, l_ref):
    x = x_ref[...]
    xw_ref[...] = jnp.dot(x, wc_ref[...], preferred_element_type=jnp.float32)
    l_ref[...] = (
        jnp.dot(x, wl_ref[...], preferred_element_type=jnp.float32)
        + bl_ref[...])


def _scale1_body(xw_ref, degp_ref, y_ref, dis_ref):
    n = y_ref.shape[0]
    deg = degp_ref[0, :n, 0:1] + degp_ref[1, :n, 0:1]
    dis = jnp.where(deg > 0.0, lax.rsqrt(jnp.where(deg > 0.0, deg, 1.0)), 0.0)
    dis_ref[...] = dis
    y_ref[...] = dis * xw_ref[...]


def _mid_body(m_ref, l1_ref, dis_ref, bc_ref, wc2_ref, wl2_ref, bl2_ref,
              y2_ref, l2_ref):
    n = l1_ref.shape[0]
    dis = dis_ref[...]
    conv1 = dis * (m_ref[0, :n, :] + m_ref[1, :n, :]) + bc_ref[...]
    h = jnp.maximum(conv1 + l1_ref[...], 0.0)
    y2_ref[...] = dis * jnp.dot(h, wc2_ref[...],
                                preferred_element_type=jnp.float32)
    l2_ref[...] = (
        jnp.dot(h, wl2_ref[...], preferred_element_type=jnp.float32)
        + bl2_ref[...])


def _final_body(m_ref, l2_ref, dis_ref, bc2_ref, grp_ref, wh_ref, bh_ref,
                out_ref):
    n = l2_ref.shape[0]
    g = out_ref.shape[0]
    h2 = (dis_ref[...] * (m_ref[0, :n, :] + m_ref[1, :n, :])
          + bc2_ref[...] + l2_ref[...])
    seg = lax.broadcasted_iota(jnp.int32, (g, n), 0)
    onehot_t = (seg == grp_ref[...]).astype(jnp.float32)
    pooled = jnp.dot(onehot_t, h2, preferred_element_type=jnp.float32)
    out_ref[...] = (
        jnp.dot(jnp.maximum(pooled, 0.0), wh_ref[...],
                preferred_element_type=jnp.float32)
        + bh_ref[...])


def kernel(x, edge_index, group, W_conv1, b_conv1, W_lin1, b_lin1,
           W_conv2, b_conv2, W_lin2, b_lin2, W_head, b_head):
    n, d = x.shape
    h = W_conv1.shape[1]
    o = W_conv2.shape[1]
    g = 256
    e = edge_index.shape[1]

    chunks = -(-e // (NW * C))
    e_pad = chunks * NW * C
    n_acc = -(-(n + 1) // (NS * 8)) * (NS * 8)

    src = edge_index[0].astype(jnp.int32)
    dst = edge_index[1].astype(jnp.int32)
    src_r = jnp.concatenate(
        [src, jnp.zeros((e_pad - e,), jnp.int32)]).reshape(NW, chunks, C)
    dst_r = jnp.concatenate(
        [dst, jnp.full((e_pad - e,), n, jnp.int32)]).reshape(NW, chunks, C)

    ones_chunk = jnp.ones((C, d), jnp.float32)
    zeros_big = jnp.zeros((n_acc, d), jnp.float32)

    deg_parts = _make_degree_kernel(n_acc, chunks, d)(dst_r, zeros_big,
                                                      ones_chunk)

    xw1, l1 = _tc_call(
        _mm1_body,
        [jax.ShapeDtypeStruct((n, h), jnp.float32),
         jax.ShapeDtypeStruct((n, h), jnp.float32)],
    )(x, W_conv1, W_lin1, b_lin1.reshape(1, h))

    y1, dis = _tc_call(
        _scale1_body,
        [jax.ShapeDtypeStruct((n, h), jnp.float32),
         jax.ShapeDtypeStruct((n, 1), jnp.float32)],
    )(xw1, deg_parts)

    mp = _make_mp_kernel(n_acc, chunks, h)
    m1 = mp(y1, src_r, dst_r, zeros_big)

    y2, l2 = _tc_call(
        _mid_body,
        [jax.ShapeDtypeStruct((n, o), jnp.float32),
         jax.ShapeDtypeStruct((n, o), jnp.float32)],
    )(m1, l1, dis, b_conv1.reshape(1, h), W_conv2, W_lin2,
      b_lin2.reshape(1, o))

    m2 = mp(y2, src_r, dst_r, zeros_big)

    wh_pad = jnp.zeros((o, 128), jnp.float32).at[:, 0].set(W_head[:, 0])
    out_pad = _tc_call(
        _final_body,
        jax.ShapeDtypeStruct((g, 128), jnp.float32),
    )(m2, l2, dis, b_conv2.reshape(1, o), group.astype(jnp.int32).reshape(1, n),
      wh_pad, b_head.reshape(1, 1))

    return out_pad[:, 0]
